# baseline (speedup 1.0000x reference)
import functools

import jax
import jax.numpy as jnp
from jax import lax
from jax.experimental import pallas as pl
from jax.experimental.pallas import tpu as pltpu
from jax.experimental.pallas import tpu_sc as plsc

_VOCAB = 21013
_EMB = 50
_EMBP = 128
_WIN = 5
_BATCH = 16384
_HID = 100

_NW = 32
_TOTAL = _BATCH * _WIN
_ROWS_PER_W = _TOTAL // _NW
_CH = 128
_NCH = _ROWS_PER_W // _CH
_GRP = 5


def _sc_gather(table, idx4d):
    mesh = plsc.VectorSubcoreMesh(core_axis_name="c", subcore_axis_name="s")

    @functools.partial(
        pl.kernel,
        mesh=mesh,
        out_type=jax.ShapeDtypeStruct((_TOTAL // _CH, _CH, _EMBP), jnp.float32),
        scratch_types=[
            pltpu.VMEM((_NCH, 1, _CH), jnp.int32),
            pltpu.VMEM((_GRP, _CH, _EMBP), jnp.float32),
            pltpu.SemaphoreType.DMA,
            pltpu.SemaphoreType.DMA,
        ],
    )
    def gather_kernel(table_hbm, idx_hbm, out_hbm, idx_v, rows_v, sem, sem2):
        wid = lax.axis_index("s") * 2 + lax.axis_index("c")
        pltpu.sync_copy(idx_hbm.at[wid], idx_v)
        for g in range(_NCH // _GRP):
            gathers = []
            for j in range(_GRP):
                c = g * _GRP + j
                gathers.append(
                    pltpu.async_copy(
                        table_hbm.at[idx_v.at[c, 0]],
                        rows_v.at[j],
                        sem,
                    )
                )
            writebacks = []
            for j in range(_GRP):
                gathers[j].wait()
                slot = wid * _NCH + g * _GRP + j
                writebacks.append(
                    pltpu.async_copy(rows_v.at[j], out_hbm.at[slot], sem2)
                )
            for wb in writebacks:
                wb.wait()

    return gather_kernel(table, idx4d)


def _mlp_kernel(x_ref, w1_ref, b1_ref, w2t_ref, b2_ref, o_ref):
    h = jnp.dot(x_ref[0], w1_ref[0], preferred_element_type=jnp.float32)
    for w in range(1, _WIN):
        h = h + jnp.dot(x_ref[w], w1_ref[w], preferred_element_type=jnp.float32)
    h = jnp.tanh(h + b1_ref[...])
    o_ref[...] = (
        jnp.dot(h, w2t_ref[...], preferred_element_type=jnp.float32)
        + b2_ref[...]
    )


def _tc_mlp(x, w1p, b1, w2t, b2):
    blk = 2048
    return pl.pallas_call(
        _mlp_kernel,
        grid=(_BATCH // blk,),
        in_specs=[
            pl.BlockSpec((_WIN, blk, _EMBP), lambda i: (0, i, 0)),
            pl.BlockSpec((_WIN, _EMBP, _HID), lambda i: (0, 0, 0)),
            pl.BlockSpec((1, _HID), lambda i: (0, 0)),
            pl.BlockSpec((_HID, _WIN), lambda i: (0, 0)),
            pl.BlockSpec((1, _WIN), lambda i: (0, 0)),
        ],
        out_specs=pl.BlockSpec((blk, _WIN), lambda i: (i, 0)),
        out_shape=jax.ShapeDtypeStruct((_BATCH, _WIN), jnp.float32),
    )(x, w1p, b1, w2t, b2)


def kernel(input, table, W1, b1, W2, b2):
    table_p = jnp.pad(table, ((0, 0), (0, _EMBP - _EMB)))
    idx4d = input.T.reshape(_NW, _NCH, 1, _CH)
    rows = _sc_gather(table_p, idx4d)
    x = rows.reshape(_WIN, _BATCH, _EMBP)
    w1p = jnp.pad(
        W1.T.reshape(_WIN, _EMB, _HID), ((0, 0), (0, _EMBP - _EMB), (0, 0))
    )
    return _tc_mlp(x, w1p, b1.reshape(1, -1), W2.T, b2.reshape(1, -1))


# speedup vs baseline: 1.0299x; 1.0299x over previous
import functools

import jax
import jax.numpy as jnp
from jax import lax
from jax.experimental import pallas as pl
from jax.experimental.pallas import tpu as pltpu
from jax.experimental.pallas import tpu_sc as plsc

_VOCAB = 21013
_EMB = 50
_EMBP = 128
_WIN = 5
_BATCH = 16384
_HID = 100

_NW = 32
_TOTAL = _BATCH * _WIN
_ROWS_PER_W = _TOTAL // _NW
_CH = 128
_NCH = _ROWS_PER_W // _CH
_PIPE = 6
_LAG = 3


def _sc_gather(table, idx4d):
    mesh = plsc.VectorSubcoreMesh(core_axis_name="c", subcore_axis_name="s")

    @functools.partial(
        pl.kernel,
        mesh=mesh,
        out_type=jax.ShapeDtypeStruct((_TOTAL // _CH, _CH, _EMBP), jnp.float32),
        scratch_types=[
            pltpu.VMEM((_NCH, 1, _CH), jnp.int32),
            pltpu.VMEM((_PIPE, _CH, _EMBP), jnp.float32),
            pltpu.SemaphoreType.DMA,
            pltpu.SemaphoreType.DMA,
        ],
    )
    def gather_kernel(table_hbm, idx_hbm, out_hbm, idx_v, rows_v, sem, sem2):
        wid = lax.axis_index("s") * 2 + lax.axis_index("c")
        pltpu.sync_copy(idx_hbm.at[wid], idx_v)
        gathers = [None] * _NCH
        writebacks = [None] * _NCH
        for c in range(_NCH + _LAG):
            if c < _NCH:
                if c >= _PIPE:
                    writebacks[c - _PIPE].wait()
                gathers[c] = pltpu.async_copy(
                    table_hbm.at[idx_v.at[c, 0]],
                    rows_v.at[c % _PIPE],
                    sem,
                )
            d = c - _LAG
            if d >= 0:
                gathers[d].wait()
                writebacks[d] = pltpu.async_copy(
                    rows_v.at[d % _PIPE], out_hbm.at[wid * _NCH + d], sem2
                )
        for d in range(_NCH - _PIPE, _NCH):
            writebacks[d].wait()

    return gather_kernel(table, idx4d)


def _mlp_kernel(x_ref, w1_ref, b1_ref, w2t_ref, b2_ref, o_ref):
    h = jnp.dot(x_ref[0], w1_ref[0], preferred_element_type=jnp.float32)
    for w in range(1, _WIN):
        h = h + jnp.dot(x_ref[w], w1_ref[w], preferred_element_type=jnp.float32)
    h = jnp.tanh(h + b1_ref[...])
    o_ref[...] = (
        jnp.dot(h, w2t_ref[...], preferred_element_type=jnp.float32)
        + b2_ref[...]
    )


def _tc_mlp(x, w1p, b1, w2t, b2):
    blk = 4096
    return pl.pallas_call(
        _mlp_kernel,
        grid=(_BATCH // blk,),
        in_specs=[
            pl.BlockSpec((_WIN, blk, _EMBP), lambda i: (0, i, 0)),
            pl.BlockSpec((_WIN, _EMBP, _HID), lambda i: (0, 0, 0)),
            pl.BlockSpec((1, _HID), lambda i: (0, 0)),
            pl.BlockSpec((_HID, _WIN), lambda i: (0, 0)),
            pl.BlockSpec((1, _WIN), lambda i: (0, 0)),
        ],
        out_specs=pl.BlockSpec((blk, _WIN), lambda i: (i, 0)),
        out_shape=jax.ShapeDtypeStruct((_BATCH, _WIN), jnp.float32),
    )(x, w1p, b1, w2t, b2)


def kernel(input, table, W1, b1, W2, b2):
    table_p = jnp.pad(table, ((0, 0), (0, _EMBP - _EMB)))
    idx4d = input.T.reshape(_NW, _NCH, 1, _CH)
    rows = _sc_gather(table_p, idx4d)
    x = rows.reshape(_WIN, _BATCH, _EMBP)
    w1p = jnp.pad(
        W1.T.reshape(_WIN, _EMB, _HID), ((0, 0), (0, _EMBP - _EMB), (0, 0))
    )
    return _tc_mlp(x, w1p, b1.reshape(1, -1), W2.T, b2.reshape(1, -1))
